# Initial kernel scaffold; baseline (speedup 1.0000x reference)
import jax
import jax.numpy as jnp
from jax.experimental import pallas as pl


def kernel(x, pos, edge_index, W_in, b_in, W_out, b_out, W_pos, b_pos, g_pos, bt_pos, W_attn, b_attn, g_attn, bt_attn, W_lin, W_src, W_dst):
    raise NotImplementedError("write your pallas kernel here")


# speedup vs baseline: 1.0550x; 1.0550x over previous
import functools

import jax
import jax.numpy as jnp
from jax import lax
from jax.experimental import pallas as pl
from jax.experimental.pallas import tpu as pltpu

N = 10000
D = 128
E_PAD = 331776
NB_T1 = 2048
N_ACC = 10368


def _t0_body(x_ref, winT, b_in, wlinT, wdstT, wsrcT, wattnT, b_attn,
             hxL_o, hxR_o, p_o, q_o):
    h = jax.nn.relu(jnp.dot(x_ref[...], winT[...],
                            preferred_element_type=jnp.float32) + b_in[...])
    hx = jnp.dot(h, wlinT[...], preferred_element_type=jnp.float32)
    p = jnp.dot(jnp.dot(h, wdstT[...], preferred_element_type=jnp.float32),
                wattnT[...], preferred_element_type=jnp.float32) + b_attn[...]
    q = jnp.dot(jnp.dot(h, wsrcT[...], preferred_element_type=jnp.float32),
                wattnT[...], preferred_element_type=jnp.float32)
    hxL_o[...] = hx[:, :64]
    hxR_o[...] = hx[:, 64:]
    p_o[...] = p
    q_o[...] = q


def _t0(x, winT, b_in, wlinT, wdstT, wsrcT, wattnT, b_attn):
    B = 2000
    grid = N // B
    full = lambda s: pl.BlockSpec(s, lambda i: (0, 0))
    return pl.pallas_call(
        _t0_body,
        grid=(grid,),
        in_specs=[pl.BlockSpec((B, D), lambda i: (i, 0)),
                  full((D, D)), full((1, D)), full((D, D)), full((D, D)),
                  full((D, D)), full((D, D)), full((1, D))],
        out_specs=[pl.BlockSpec((B, 64), lambda i: (i, 0)),
                   pl.BlockSpec((B, 64), lambda i: (i, 0)),
                   pl.BlockSpec((B, D), lambda i: (i, 0)),
                   pl.BlockSpec((B, D), lambda i: (i, 0))],
        out_shape=[jax.ShapeDtypeStruct((N, 64), jnp.float32),
                   jax.ShapeDtypeStruct((N, 64), jnp.float32),
                   jax.ShapeDtypeStruct((N, D), jnp.float32),
                   jax.ShapeDtypeStruct((N, D), jnp.float32)],
    )(x, winT, b_in, wlinT, wdstT, wsrcT, wattnT, b_attn)


def _t1_body(pqL, pqR, pw, a_ref, c0_ref, wattnT, alL_o, alR_o, dlL_o, dlR_o, st_o):
    i = pl.program_id(0)
    a = a_ref[...]
    delta = jax.nn.relu(pw[:, 0:1] * a[0:1, :] + pw[:, 1:2] * a[1:2, :]
                        + pw[:, 2:3] * a[2:3, :] + c0_ref[...])
    pq = jnp.concatenate([pqL[...], pqR[...]], axis=1)
    alpha = pq + jnp.dot(delta, wattnT[...], preferred_element_type=jnp.float32)
    alL_o[...] = alpha[:, :64]
    alR_o[...] = alpha[:, 64:]
    dlL_o[...] = delta[:, :64]
    dlR_o[...] = delta[:, 64:]
    w = pw[:, 3:4]
    s0 = jnp.sum(w * alpha, axis=0)[None, :]
    s1 = jnp.sum(w * alpha * alpha, axis=0)[None, :]
    mask = w > 0.0
    mx = jnp.max(jnp.where(mask, alpha, -3e38), axis=0)[None, :]
    mn = jnp.min(jnp.where(mask, alpha, 3e38), axis=0)[None, :]

    @pl.when(i == 0)
    def _init():
        st_o[...] = jnp.concatenate(
            [jnp.zeros((2, D), jnp.float32),
             jnp.full((1, D), -3e38, jnp.float32),
             jnp.full((1, D), 3e38, jnp.float32),
             jnp.zeros((4, D), jnp.float32)], axis=0)

    acc = st_o[...]
    st_o[...] = jnp.concatenate(
        [acc[0:1] + s0, acc[1:2] + s1,
         jnp.maximum(acc[2:3], mx), jnp.minimum(acc[3:4], mn), acc[4:8]], axis=0)


def _t1(pqL, pqR, pw, A3, c0, wattnT):
    B = NB_T1
    grid = E_PAD // B
    full = lambda s: pl.BlockSpec(s, lambda i: (0, 0))
    eb64 = pl.BlockSpec((B, 64), lambda i: (i, 0))
    return pl.pallas_call(
        _t1_body,
        grid=(grid,),
        in_specs=[eb64, eb64, pl.BlockSpec((B, 4), lambda i: (i, 0)),
                  full((3, D)), full((1, D)), full((D, D))],
        out_specs=[eb64, eb64, eb64, eb64, full((8, D))],
        out_shape=[jax.ShapeDtypeStruct((E_PAD, 64), jnp.float32),
                   jax.ShapeDtypeStruct((E_PAD, 64), jnp.float32),
                   jax.ShapeDtypeStruct((E_PAD, 64), jnp.float32),
                   jax.ShapeDtypeStruct((E_PAD, 64), jnp.float32),
                   jax.ShapeDtypeStruct((8, D), jnp.float32)],
    )(pqL, pqR, pw, A3, c0, wattnT)


def _t2_body(nF, aF, woutT, b_out, o_ref):
    outp = nF[...] / (aF[...] + 1e-16)
    o_ref[...] = jax.nn.relu(
        jnp.dot(outp, woutT[...], preferred_element_type=jnp.float32) + b_out[...])


def _t2(numerF, asumF, woutT, b_out):
    B = 2000
    grid = N // B
    full = lambda s: pl.BlockSpec(s, lambda i: (0, 0))
    return pl.pallas_call(
        _t2_body,
        grid=(grid,),
        in_specs=[pl.BlockSpec((B, D), lambda i: (i, 0)),
                  pl.BlockSpec((B, D), lambda i: (i, 0)),
                  full((D, D)), full((1, D))],
        out_specs=pl.BlockSpec((B, D), lambda i: (i, 0)),
        out_shape=jax.ShapeDtypeStruct((N, D), jnp.float32),
    )(numerF, asumF, woutT, b_out)


def kernel(x, pos, edge_index, W_in, b_in, W_out, b_out, W_pos, b_pos, g_pos,
           bt_pos, W_attn, b_attn, g_attn, bt_attn, W_lin, W_src, W_dst):
    E_orig = edge_index.shape[1]
    E_tot = E_orig + N
    pad = E_PAD - E_tot

    src_o, dst_o = edge_index[0], edge_index[1]
    keep_o = src_o != dst_o
    loop = jnp.arange(N, dtype=jnp.int32)
    src = jnp.concatenate([src_o, loop, jnp.zeros((pad,), jnp.int32)])
    dst = jnp.concatenate([jnp.where(keep_o, dst_o, N), loop,
                           jnp.full((pad,), N, jnp.int32)])
    w = jnp.concatenate([keep_o.astype(jnp.float32), jnp.ones((N,), jnp.float32),
                         jnp.zeros((pad,), jnp.float32)])
    dstc = jnp.minimum(dst, N - 1)

    hxL, hxR, P, Q = _t0(x, W_in.T, b_in[None, :], W_lin.T, W_dst.T, W_src.T,
                         W_attn.T, b_attn[None, :])

    pdiff = pos[dstc] - pos[src]
    pqF = P[dstc] - Q[src]
    pqL, pqR = pqF[:, :64], pqF[:, 64:]
    cnt = jnp.sum(w)
    S1v = jnp.sum(w[:, None] * pdiff, axis=0)
    S2m = (w[:, None] * pdiff).T @ pdiff

    pbar = S1v / cnt
    M = S2m / cnt - jnp.outer(pbar, pbar)
    var_pos = jnp.einsum("ca,ab,cb->c", W_pos, M, W_pos)
    s_pos = g_pos / jnp.sqrt(var_pos + 1e-5)
    A3 = W_pos.T * s_pos[None, :]
    c0 = bt_pos - (W_pos @ pbar) * s_pos

    pw = jnp.concatenate([pdiff, w[:, None]], axis=1)

    alL, alR, dlL, dlR, st = _t1(pqL, pqR, pw, A3, c0[None, :], W_attn.T)

    mu = st[0] / cnt
    var_a = st[1] / cnt - mu * mu
    s_att = g_attn / jnp.sqrt(var_a + 1e-5)
    t0 = bt_attn - mu * s_att
    gmax = jax.nn.relu(jnp.maximum(s_att * st[2], s_att * st[3]) + t0)

    alpha_pre = jnp.concatenate([alL, alR], axis=1)
    delta = jnp.concatenate([dlL, dlR], axis=1)
    ae = jnp.exp(jax.nn.relu(alpha_pre * s_att + t0) - gmax)
    msg = ae * (jnp.concatenate([hxL, hxR], axis=1)[src] + delta)
    numer = jax.ops.segment_sum(msg, dst, num_segments=N + 1)
    asum = jax.ops.segment_sum(ae, dst, num_segments=N + 1)
    numerF = numer[:N]
    asumF = asum[:N]

    return _t2(numerF, asumF, W_out.T, b_out[None, :])


# speedup vs baseline: 2.4096x; 2.2839x over previous
import functools

import jax
import jax.numpy as jnp
from jax import lax
from jax.experimental import pallas as pl
from jax.experimental.pallas import tpu as pltpu
from jax.experimental.pallas import tpu_sc as plsc

N = 10000
D = 128
E_PAD = 331776
NB_T1 = 2048
N_ACC = 10016
KC = 128
NW = 32
RPS = N_ACC // 16


def _s1_body(src_h, dstc_h, w_h, px_h, py_h, pz_h, p_hbm, q_hbm,
             pqL_h, pqR_h, pdx_h, pdy_h, pdz_h, mom_h,
             px_v, py_v, pz_v, src_v, dst_v, w_v, prow, qrow,
             pqL_v, pqR_v, pdx_v, pdy_v, pdz_v, mom_v, sem):
    c = lax.axis_index("c")
    s = lax.axis_index("s")
    wid = s * 2 + c
    chunk_e = E_PAD // NW
    base0 = wid * chunk_e
    pltpu.sync_copy(px_h, px_v)
    pltpu.sync_copy(py_h, py_v)
    pltpu.sync_copy(pz_h, pz_v)

    def chunk(j, moms):
        base = base0 + j * KC
        c1 = pltpu.async_copy(src_h.at[pl.ds(base, KC)], src_v, sem)
        c2 = pltpu.async_copy(dstc_h.at[pl.ds(base, KC)], dst_v, sem)
        c3 = pltpu.async_copy(w_h.at[pl.ds(base, KC)], w_v, sem)
        c1.wait()
        c2.wait()
        c3.wait()
        g1 = pltpu.async_copy(p_hbm.at[dst_v], prow, sem)
        g2 = pltpu.async_copy(q_hbm.at[src_v], qrow, sem)
        g1.wait()
        g2.wait()

        def prow_loop(i, carry):
            for cb in range(8):
                r = prow[i, pl.ds(cb * 16, 16)] - qrow[i, pl.ds(cb * 16, 16)]
                if cb < 4:
                    pqL_v[i, pl.ds(cb * 16, 16)] = r
                else:
                    pqR_v[i, pl.ds((cb - 4) * 16, 16)] = r
            return carry

        lax.fori_loop(0, KC, prow_loop, 0)
        w1 = pltpu.async_copy(pqL_v, pqL_h.at[pl.ds(base, KC)], sem)
        w2 = pltpu.async_copy(pqR_v, pqR_h.at[pl.ds(base, KC)], sem)

        def pos_loop(g, m):
            sl = pl.ds(g * 16, 16)
            si = src_v[sl]
            di = dst_v[sl]
            wv = w_v[sl]
            dx = plsc.load_gather(px_v, [di]) - plsc.load_gather(px_v, [si])
            dy = plsc.load_gather(py_v, [di]) - plsc.load_gather(py_v, [si])
            dz = plsc.load_gather(pz_v, [di]) - plsc.load_gather(pz_v, [si])
            pdx_v[sl] = dx
            pdy_v[sl] = dy
            pdz_v[sl] = dz
            wdx = wv * dx
            wdy = wv * dy
            wdz = wv * dz
            return (m[0] + wdx, m[1] + wdy, m[2] + wdz,
                    m[3] + wdx * dx, m[4] + wdy * dy, m[5] + wdz * dz,
                    m[6] + wdx * dy, m[7] + wdx * dz, m[8] + wdy * dz,
                    m[9] + wv)

        moms = lax.fori_loop(0, KC // 16, pos_loop, moms)
        w3 = pltpu.async_copy(pdx_v, pdx_h.at[pl.ds(base, KC)], sem)
        w4 = pltpu.async_copy(pdy_v, pdy_h.at[pl.ds(base, KC)], sem)
        w5 = pltpu.async_copy(pdz_v, pdz_h.at[pl.ds(base, KC)], sem)
        w1.wait()
        w2.wait()
        w3.wait()
        w4.wait()
        w5.wait()
        return moms

    zero = jnp.zeros((16,), jnp.float32)
    moms = lax.fori_loop(0, chunk_e // KC, chunk, tuple(zero for _ in range(10)))
    for r in range(10):
        mom_v[r, :] = moms[r]
    for r in range(10, 16):
        mom_v[r, :] = zero
    pltpu.sync_copy(mom_v, mom_h.at[wid])


def _s1(src, dstc, w, posx, posy, posz, P, Q):
    mesh = plsc.VectorSubcoreMesh(core_axis_name="c", subcore_axis_name="s")
    f32 = jnp.float32
    k = functools.partial(
        pl.kernel, mesh=mesh,
        out_type=[jax.ShapeDtypeStruct((E_PAD, 64), f32),
                  jax.ShapeDtypeStruct((E_PAD, 64), f32),
                  jax.ShapeDtypeStruct((E_PAD,), f32),
                  jax.ShapeDtypeStruct((E_PAD,), f32),
                  jax.ShapeDtypeStruct((E_PAD,), f32),
                  jax.ShapeDtypeStruct((NW, 16, 16), f32)],
        scratch_types=[pltpu.VMEM((N,), f32), pltpu.VMEM((N,), f32),
                       pltpu.VMEM((N,), f32),
                       pltpu.VMEM((KC,), jnp.int32), pltpu.VMEM((KC,), jnp.int32),
                       pltpu.VMEM((KC,), f32),
                       pltpu.VMEM((KC, D), f32), pltpu.VMEM((KC, D), f32),
                       pltpu.VMEM((KC, 64), f32), pltpu.VMEM((KC, 64), f32),
                       pltpu.VMEM((KC,), f32), pltpu.VMEM((KC,), f32),
                       pltpu.VMEM((KC,), f32), pltpu.VMEM((16, 16), f32),
                       pltpu.SemaphoreType.DMA],
        compiler_params=pltpu.CompilerParams(needs_layout_passes=False,
                                             use_tc_tiling_on_sc=False),
    )(_s1_body)
    return k(src, dstc, w, posx, posy, posz, P, Q)


def _s2_body(alL, alR, dlL, dlR, hxL, hxR, src_h, dst_h, coef_h,
             acc_out,
             a_v, d_v, hx_a, hx_b, mm_v, src_a, src_b, dst_a, dst_b, coef_v,
             acc, sem_l, sem_ga, sem_gb, sem_s):
    c = lax.axis_index("c")
    s = lax.axis_index("s")

    def zb(i, carry):
        for cb in range(8):
            mm_v[i, pl.ds(cb * 16, 16)] = jnp.zeros((16,), jnp.float32)
        return carry

    lax.fori_loop(0, KC, zb, 0)
    row0 = s * RPS

    def zc(j, carry):
        pltpu.sync_copy(mm_v, acc.at[pl.ds(row0 + j * KC, KC)])
        return carry

    lax.fori_loop(0, 4, zc, 0)
    pltpu.sync_copy(mm_v.at[pl.ds(0, RPS - 4 * KC)],
                    acc.at[pl.ds(row0 + 4 * KC, RPS - 4 * KC)])
    pltpu.sync_copy(coef_h.at[c], coef_v)
    plsc.subcore_barrier()

    svec = [coef_v[0, pl.ds(cb * 16, 16)] for cb in range(4)]
    tvec = [coef_v[1, pl.ds(cb * 16, 16)] for cb in range(4)]
    gvec = [coef_v[2, pl.ds(cb * 16, 16)] for cb in range(4)]

    chunk_e = E_PAD // 16
    NCH = chunk_e // KC
    base0 = s * chunk_e

    def issue_gather(src_buf, hx_buf, sem_g):
        @pl.when(c == 0)
        def _():
            pltpu.async_copy(hxL.at[src_buf], hx_buf, sem_g)

        @pl.when(c == 1)
        def _():
            pltpu.async_copy(hxR.at[src_buf], hx_buf, sem_g)

    def issue_linear(base, dst_buf):
        pltpu.async_copy(dst_h.at[pl.ds(base, KC)], dst_buf, sem_l)

        @pl.when(c == 0)
        def _():
            pltpu.async_copy(alL.at[pl.ds(base, KC)], a_v, sem_l)
            pltpu.async_copy(dlL.at[pl.ds(base, KC)], d_v, sem_l)

        @pl.when(c == 1)
        def _():
            pltpu.async_copy(alR.at[pl.ds(base, KC)], a_v, sem_l)
            pltpu.async_copy(dlR.at[pl.ds(base, KC)], d_v, sem_l)

    def wait_linear(dst_buf):
        pltpu.make_async_copy(dst_h.at[pl.ds(0, KC)], dst_buf, sem_l).wait()
        pltpu.make_async_copy(alL.at[pl.ds(0, KC)], a_v, sem_l).wait()
        pltpu.make_async_copy(dlL.at[pl.ds(0, KC)], d_v, sem_l).wait()

    def wait_gather(hx_buf, sem_g):
        pltpu.make_async_copy(alL.at[pl.ds(0, KC)], hx_buf, sem_g).wait()

    def wait_scatter():
        pltpu.make_async_copy(acc_out.at[0, pl.ds(0, KC)], mm_v, sem_s).wait()

    def edge_loop(hx_buf):
        def edge(i, carry2):
            for cb in range(4):
                sl = pl.ds(cb * 16, 16)
                z = jnp.maximum(a_v[i, sl] * svec[cb] + tvec[cb], 0.0)
                ae = jnp.exp(z - gvec[cb])
                mm_v[i, pl.ds(64 + cb * 16, 16)] = ae
                mm_v[i, sl] = ae * (hx_buf[i, sl] + d_v[i, sl])
            return carry2

        lax.fori_loop(0, KC, edge, 0, unroll=2)

    pltpu.sync_copy(src_h.at[pl.ds(base0, KC)], src_a)
    issue_gather(src_a, hx_a, sem_ga)

    def iot(g, carry):
        dst_a[pl.ds(g * 16, 16)] = lax.iota(jnp.int32, 16) + g * 16
        return carry

    lax.fori_loop(0, KC // 16, iot, 0)
    pltpu.async_copy(mm_v, acc.at[dst_a], sem_s, add=True)

    def pair(t, carry):
        j = 2 * t
        issue_linear(base0 + j * KC, dst_a)
        pltpu.sync_copy(src_h.at[pl.ds(base0 + (j + 1) * KC, KC)], src_b)
        issue_gather(src_b, hx_b, sem_gb)
        wait_linear(dst_a)
        wait_scatter()
        wait_gather(hx_a, sem_ga)
        edge_loop(hx_a)
        pltpu.async_copy(mm_v, acc.at[dst_a], sem_s, add=True)
        j1 = j + 1
        issue_linear(base0 + j1 * KC, dst_b)
        basen = jnp.minimum(base0 + (j1 + 1) * KC, base0 + (NCH - 1) * KC)
        pltpu.sync_copy(src_h.at[pl.ds(basen, KC)], src_a)
        issue_gather(src_a, hx_a, sem_ga)
        wait_linear(dst_b)
        wait_scatter()
        wait_gather(hx_b, sem_gb)
        edge_loop(hx_b)
        pltpu.async_copy(mm_v, acc.at[dst_b], sem_s, add=True)
        return carry

    lax.fori_loop(0, NCH // 2, pair, 0)
    wait_scatter()
    wait_gather(hx_a, sem_ga)
    plsc.subcore_barrier()
    pltpu.sync_copy(acc.at[pl.ds(row0, RPS)], acc_out.at[c, pl.ds(row0, RPS)])


def _s2(alL, alR, dlL, dlR, hxL, hxR, src, dst, coef):
    mesh = plsc.VectorSubcoreMesh(core_axis_name="c", subcore_axis_name="s")
    f32 = jnp.float32
    k = functools.partial(
        pl.kernel, mesh=mesh,
        out_type=[jax.ShapeDtypeStruct((2, N_ACC, D), f32)],
        scratch_types=[pltpu.VMEM((KC, 64), f32), pltpu.VMEM((KC, 64), f32),
                       pltpu.VMEM((KC, 64), f32), pltpu.VMEM((KC, 64), f32),
                       pltpu.VMEM((KC, D), f32),
                       pltpu.VMEM((KC,), jnp.int32), pltpu.VMEM((KC,), jnp.int32),
                       pltpu.VMEM((KC,), jnp.int32), pltpu.VMEM((KC,), jnp.int32),
                       pltpu.VMEM((3, 64), f32),
                       pltpu.VMEM_SHARED((N_ACC, D), f32),
                       pltpu.SemaphoreType.DMA, pltpu.SemaphoreType.DMA,
                       pltpu.SemaphoreType.DMA, pltpu.SemaphoreType.DMA],
        compiler_params=pltpu.CompilerParams(needs_layout_passes=False,
                                             use_tc_tiling_on_sc=False),
    )(_s2_body)
    return k(alL, alR, dlL, dlR, hxL, hxR, src, dst, coef)


def _t0_body(x_ref, winT, b_in, wlinT, wdstT, wsrcT, wattnT, b_attn,
             hx_o, p_o, q_o):
    h = jax.nn.relu(jnp.dot(x_ref[...], winT[...],
                            preferred_element_type=jnp.float32) + b_in[...])
    hx = jnp.dot(h, wlinT[...], preferred_element_type=jnp.float32)
    p = jnp.dot(jnp.dot(h, wdstT[...], preferred_element_type=jnp.float32),
                wattnT[...], preferred_element_type=jnp.float32) + b_attn[...]
    q = jnp.dot(jnp.dot(h, wsrcT[...], preferred_element_type=jnp.float32),
                wattnT[...], preferred_element_type=jnp.float32)
    hx_o[...] = hx
    p_o[...] = p
    q_o[...] = q


def _t0(x, winT, b_in, wlinT, wdstT, wsrcT, wattnT, b_attn):
    B = 2000
    grid = N // B
    full = lambda s: pl.BlockSpec(s, lambda i: (0, 0))
    return pl.pallas_call(
        _t0_body,
        grid=(grid,),
        in_specs=[pl.BlockSpec((B, D), lambda i: (i, 0)),
                  full((D, D)), full((1, D)), full((D, D)), full((D, D)),
                  full((D, D)), full((D, D)), full((1, D))],
        out_specs=[pl.BlockSpec((B, D), lambda i: (i, 0)),
                   pl.BlockSpec((B, D), lambda i: (i, 0)),
                   pl.BlockSpec((B, D), lambda i: (i, 0))],
        out_shape=[jax.ShapeDtypeStruct((N, D), jnp.float32),
                   jax.ShapeDtypeStruct((N, D), jnp.float32),
                   jax.ShapeDtypeStruct((N, D), jnp.float32)],
    )(x, winT, b_in, wlinT, wdstT, wsrcT, wattnT, b_attn)


def _t1_body(pqL, pqR, pw, a_ref, c0_ref, wattnT, alL_o, alR_o, dlL_o, dlR_o, st_o):
    i = pl.program_id(0)
    a = a_ref[...]
    delta = jax.nn.relu(pw[:, 0:1] * a[0:1, :] + pw[:, 1:2] * a[1:2, :]
                        + pw[:, 2:3] * a[2:3, :] + c0_ref[...])
    pq = jnp.concatenate([pqL[...], pqR[...]], axis=1)
    alpha = pq + jnp.dot(delta, wattnT[...], preferred_element_type=jnp.float32)
    alL_o[...] = alpha[:, :64]
    alR_o[...] = alpha[:, 64:]
    dlL_o[...] = delta[:, :64]
    dlR_o[...] = delta[:, 64:]
    w = pw[:, 3:4]
    s0 = jnp.sum(w * alpha, axis=0)[None, :]
    s1 = jnp.sum(w * alpha * alpha, axis=0)[None, :]
    mask = w > 0.0
    mx = jnp.max(jnp.where(mask, alpha, -3e38), axis=0)[None, :]
    mn = jnp.min(jnp.where(mask, alpha, 3e38), axis=0)[None, :]

    @pl.when(i == 0)
    def _init():
        st_o[...] = jnp.concatenate(
            [jnp.zeros((2, D), jnp.float32),
             jnp.full((1, D), -3e38, jnp.float32),
             jnp.full((1, D), 3e38, jnp.float32),
             jnp.zeros((4, D), jnp.float32)], axis=0)

    acc = st_o[...]
    st_o[...] = jnp.concatenate(
        [acc[0:1] + s0, acc[1:2] + s1,
         jnp.maximum(acc[2:3], mx), jnp.minimum(acc[3:4], mn), acc[4:8]], axis=0)


def _t1(pqL, pqR, pw, A3, c0, wattnT):
    B = NB_T1
    grid = E_PAD // B
    full = lambda s: pl.BlockSpec(s, lambda i: (0, 0))
    eb64 = pl.BlockSpec((B, 64), lambda i: (i, 0))
    return pl.pallas_call(
        _t1_body,
        grid=(grid,),
        in_specs=[eb64, eb64, pl.BlockSpec((B, 4), lambda i: (i, 0)),
                  full((3, D)), full((1, D)), full((D, D))],
        out_specs=[eb64, eb64, eb64, eb64, full((8, D))],
        out_shape=[jax.ShapeDtypeStruct((E_PAD, 64), jnp.float32),
                   jax.ShapeDtypeStruct((E_PAD, 64), jnp.float32),
                   jax.ShapeDtypeStruct((E_PAD, 64), jnp.float32),
                   jax.ShapeDtypeStruct((E_PAD, 64), jnp.float32),
                   jax.ShapeDtypeStruct((8, D), jnp.float32)],
    )(pqL, pqR, pw, A3, c0, wattnT)


def _t2_body(nF, aF, woutT, b_out, o_ref):
    outp = nF[...] / (aF[...] + 1e-16)
    o_ref[...] = jax.nn.relu(
        jnp.dot(outp, woutT[...], preferred_element_type=jnp.float32) + b_out[...])


def _t2(numerF, asumF, woutT, b_out):
    B = 2000
    grid = N // B
    full = lambda s: pl.BlockSpec(s, lambda i: (0, 0))
    return pl.pallas_call(
        _t2_body,
        grid=(grid,),
        in_specs=[pl.BlockSpec((B, D), lambda i: (i, 0)),
                  pl.BlockSpec((B, D), lambda i: (i, 0)),
                  full((D, D)), full((1, D))],
        out_specs=pl.BlockSpec((B, D), lambda i: (i, 0)),
        out_shape=jax.ShapeDtypeStruct((N, D), jnp.float32),
    )(numerF, asumF, woutT, b_out)


def kernel(x, pos, edge_index, W_in, b_in, W_out, b_out, W_pos, b_pos, g_pos,
           bt_pos, W_attn, b_attn, g_attn, bt_attn, W_lin, W_src, W_dst):
    E_orig = edge_index.shape[1]
    E_tot = E_orig + N
    pad = E_PAD - E_tot

    src_o, dst_o = edge_index[0], edge_index[1]
    keep_o = src_o != dst_o
    loop = jnp.arange(N, dtype=jnp.int32)
    src = jnp.concatenate([src_o, loop, jnp.zeros((pad,), jnp.int32)])
    dst = jnp.concatenate([jnp.where(keep_o, dst_o, N), loop,
                           jnp.full((pad,), N, jnp.int32)])
    w = jnp.concatenate([keep_o.astype(jnp.float32), jnp.ones((N,), jnp.float32),
                         jnp.zeros((pad,), jnp.float32)])
    dstc = jnp.minimum(dst, N - 1)

    hx, P, Q = _t0(x, W_in.T, b_in[None, :], W_lin.T, W_dst.T, W_src.T,
                   W_attn.T, b_attn[None, :])

    posx, posy, posz = pos[:, 0], pos[:, 1], pos[:, 2]
    pqL, pqR, pdx, pdy, pdz, mom = _s1(src, dstc, w, posx, posy, posz, P, Q)
    msum = jnp.sum(mom, axis=(0, 2))
    cnt = msum[9]
    S1v = msum[0:3]
    S2m = jnp.stack([jnp.stack([msum[3], msum[6], msum[7]]),
                     jnp.stack([msum[6], msum[4], msum[8]]),
                     jnp.stack([msum[7], msum[8], msum[5]])])

    pbar = S1v / cnt
    M = S2m / cnt - jnp.outer(pbar, pbar)
    var_pos = jnp.einsum("ca,ab,cb->c", W_pos, M, W_pos)
    s_pos = g_pos / jnp.sqrt(var_pos + 1e-5)
    A3 = W_pos.T * s_pos[None, :]
    c0 = bt_pos - (W_pos @ pbar) * s_pos

    pw = jnp.stack([pdx, pdy, pdz, w], axis=1)

    alL, alR, dlL, dlR, st = _t1(pqL, pqR, pw, A3, c0[None, :], W_attn.T)

    mu = st[0] / cnt
    var_a = st[1] / cnt - mu * mu
    s_att = g_attn / jnp.sqrt(var_a + 1e-5)
    t0 = bt_attn - mu * s_att
    gmax = jax.nn.relu(jnp.maximum(s_att * st[2], s_att * st[3]) + t0)

    coef = jnp.stack([jnp.stack([s_att[:64], t0[:64], gmax[:64]]),
                      jnp.stack([s_att[64:], t0[64:], gmax[64:]])])
    hxL = hx[:, :64] + 0.0
    hxR = hx[:, 64:] + 0.0
    (acc,) = _s2(alL, alR, dlL, dlR, hxL, hxR, src, dst, coef)
    numerF = jnp.concatenate([acc[0, :N, :64], acc[1, :N, :64]], axis=1)
    asumF = jnp.concatenate([acc[0, :N, 64:], acc[1, :N, 64:]], axis=1)

    return _t2(numerF, asumF, W_out.T, b_out[None, :])


# speedup vs baseline: 2.4405x; 1.0128x over previous
import functools

import jax
import jax.numpy as jnp
from jax import lax
from jax.experimental import pallas as pl
from jax.experimental.pallas import tpu as pltpu
from jax.experimental.pallas import tpu_sc as plsc

N = 10000
D = 128
E_PAD = 331776
NB_T1 = 4096
N_ACC = 10016
KC = 128
NW = 32
RPS = N_ACC // 16


def _s1_body(src_h, dstc_h, w_h, px_h, py_h, pz_h, p_hbm, q_hbm,
             pqL_h, pqR_h, pdx_h, pdy_h, pdz_h, mom_h,
             px_v, py_v, pz_v, src_v, dst_v, w_v, prow, qrow,
             pqL_v, pqR_v, pdx_v, pdy_v, pdz_v, mom_v, sem):
    c = lax.axis_index("c")
    s = lax.axis_index("s")
    wid = s * 2 + c
    chunk_e = E_PAD // NW
    base0 = wid * chunk_e
    pltpu.sync_copy(px_h, px_v)
    pltpu.sync_copy(py_h, py_v)
    pltpu.sync_copy(pz_h, pz_v)

    def chunk(j, moms):
        base = base0 + j * KC
        c1 = pltpu.async_copy(src_h.at[pl.ds(base, KC)], src_v, sem)
        c2 = pltpu.async_copy(dstc_h.at[pl.ds(base, KC)], dst_v, sem)
        c3 = pltpu.async_copy(w_h.at[pl.ds(base, KC)], w_v, sem)
        c1.wait()
        c2.wait()
        c3.wait()
        g1 = pltpu.async_copy(p_hbm.at[dst_v], prow, sem)
        g2 = pltpu.async_copy(q_hbm.at[src_v], qrow, sem)
        g1.wait()
        g2.wait()

        def prow_loop(i, carry):
            for cb in range(8):
                r = prow[i, pl.ds(cb * 16, 16)] - qrow[i, pl.ds(cb * 16, 16)]
                if cb < 4:
                    pqL_v[i, pl.ds(cb * 16, 16)] = r
                else:
                    pqR_v[i, pl.ds((cb - 4) * 16, 16)] = r
            return carry

        lax.fori_loop(0, KC, prow_loop, 0)
        w1 = pltpu.async_copy(pqL_v, pqL_h.at[pl.ds(base, KC)], sem)
        w2 = pltpu.async_copy(pqR_v, pqR_h.at[pl.ds(base, KC)], sem)

        def pos_loop(g, m):
            sl = pl.ds(g * 16, 16)
            si = src_v[sl]
            di = dst_v[sl]
            wv = w_v[sl]
            dx = plsc.load_gather(px_v, [di]) - plsc.load_gather(px_v, [si])
            dy = plsc.load_gather(py_v, [di]) - plsc.load_gather(py_v, [si])
            dz = plsc.load_gather(pz_v, [di]) - plsc.load_gather(pz_v, [si])
            pdx_v[sl] = dx
            pdy_v[sl] = dy
            pdz_v[sl] = dz
            wdx = wv * dx
            wdy = wv * dy
            wdz = wv * dz
            return (m[0] + wdx, m[1] + wdy, m[2] + wdz,
                    m[3] + wdx * dx, m[4] + wdy * dy, m[5] + wdz * dz,
                    m[6] + wdx * dy, m[7] + wdx * dz, m[8] + wdy * dz,
                    m[9] + wv)

        moms = lax.fori_loop(0, KC // 16, pos_loop, moms)
        w3 = pltpu.async_copy(pdx_v, pdx_h.at[pl.ds(base, KC)], sem)
        w4 = pltpu.async_copy(pdy_v, pdy_h.at[pl.ds(base, KC)], sem)
        w5 = pltpu.async_copy(pdz_v, pdz_h.at[pl.ds(base, KC)], sem)
        w1.wait()
        w2.wait()
        w3.wait()
        w4.wait()
        w5.wait()
        return moms

    zero = jnp.zeros((16,), jnp.float32)
    moms = lax.fori_loop(0, chunk_e // KC, chunk, tuple(zero for _ in range(10)))
    for r in range(10):
        mom_v[r, :] = moms[r]
    for r in range(10, 16):
        mom_v[r, :] = zero
    pltpu.sync_copy(mom_v, mom_h.at[wid])


def _s1(src, dstc, w, posx, posy, posz, P, Q):
    mesh = plsc.VectorSubcoreMesh(core_axis_name="c", subcore_axis_name="s")
    f32 = jnp.float32
    k = functools.partial(
        pl.kernel, mesh=mesh,
        out_type=[jax.ShapeDtypeStruct((E_PAD, 64), f32),
                  jax.ShapeDtypeStruct((E_PAD, 64), f32),
                  jax.ShapeDtypeStruct((E_PAD,), f32),
                  jax.ShapeDtypeStruct((E_PAD,), f32),
                  jax.ShapeDtypeStruct((E_PAD,), f32),
                  jax.ShapeDtypeStruct((NW, 16, 16), f32)],
        scratch_types=[pltpu.VMEM((N,), f32), pltpu.VMEM((N,), f32),
                       pltpu.VMEM((N,), f32),
                       pltpu.VMEM((KC,), jnp.int32), pltpu.VMEM((KC,), jnp.int32),
                       pltpu.VMEM((KC,), f32),
                       pltpu.VMEM((KC, D), f32), pltpu.VMEM((KC, D), f32),
                       pltpu.VMEM((KC, 64), f32), pltpu.VMEM((KC, 64), f32),
                       pltpu.VMEM((KC,), f32), pltpu.VMEM((KC,), f32),
                       pltpu.VMEM((KC,), f32), pltpu.VMEM((16, 16), f32),
                       pltpu.SemaphoreType.DMA],
        compiler_params=pltpu.CompilerParams(needs_layout_passes=False,
                                             use_tc_tiling_on_sc=False),
    )(_s1_body)
    return k(src, dstc, w, posx, posy, posz, P, Q)


def _s2_body(alL, alR, dlL, dlR, hxL, hxR, src_h, dst_h, coef_h,
             acc_out,
             a_v, d_v, hx_a, hx_b, mm_v, src_a, src_b, dst_a, dst_b, coef_v,
             acc, sem_l, sem_ga, sem_gb, sem_s):
    c = lax.axis_index("c")
    s = lax.axis_index("s")

    def zb(i, carry):
        for cb in range(8):
            mm_v[i, pl.ds(cb * 16, 16)] = jnp.zeros((16,), jnp.float32)
        return carry

    lax.fori_loop(0, KC, zb, 0)
    row0 = s * RPS

    def zc(j, carry):
        pltpu.sync_copy(mm_v, acc.at[pl.ds(row0 + j * KC, KC)])
        return carry

    lax.fori_loop(0, 4, zc, 0)
    pltpu.sync_copy(mm_v.at[pl.ds(0, RPS - 4 * KC)],
                    acc.at[pl.ds(row0 + 4 * KC, RPS - 4 * KC)])
    pltpu.sync_copy(coef_h.at[c], coef_v)
    plsc.subcore_barrier()

    svec = [coef_v[0, pl.ds(cb * 16, 16)] for cb in range(4)]
    tvec = [coef_v[1, pl.ds(cb * 16, 16)] for cb in range(4)]
    gvec = [coef_v[2, pl.ds(cb * 16, 16)] for cb in range(4)]

    chunk_e = E_PAD // 16
    NCH = chunk_e // KC
    base0 = s * chunk_e

    def issue_gather(src_buf, hx_buf, sem_g):
        @pl.when(c == 0)
        def _():
            pltpu.async_copy(hxL.at[src_buf], hx_buf, sem_g)

        @pl.when(c == 1)
        def _():
            pltpu.async_copy(hxR.at[src_buf], hx_buf, sem_g)

    def issue_linear(base, dst_buf):
        pltpu.async_copy(dst_h.at[pl.ds(base, KC)], dst_buf, sem_l)

        @pl.when(c == 0)
        def _():
            pltpu.async_copy(alL.at[pl.ds(base, KC)], a_v, sem_l)
            pltpu.async_copy(dlL.at[pl.ds(base, KC)], d_v, sem_l)

        @pl.when(c == 1)
        def _():
            pltpu.async_copy(alR.at[pl.ds(base, KC)], a_v, sem_l)
            pltpu.async_copy(dlR.at[pl.ds(base, KC)], d_v, sem_l)

    def wait_linear(dst_buf):
        pltpu.make_async_copy(dst_h.at[pl.ds(0, KC)], dst_buf, sem_l).wait()
        pltpu.make_async_copy(alL.at[pl.ds(0, KC)], a_v, sem_l).wait()
        pltpu.make_async_copy(dlL.at[pl.ds(0, KC)], d_v, sem_l).wait()

    def wait_gather(hx_buf, sem_g):
        pltpu.make_async_copy(alL.at[pl.ds(0, KC)], hx_buf, sem_g).wait()

    def wait_scatter():
        pltpu.make_async_copy(acc_out.at[0, pl.ds(0, KC)], mm_v, sem_s).wait()

    def edge_loop(hx_buf):
        def edge(i, carry2):
            for cb in range(4):
                sl = pl.ds(cb * 16, 16)
                z = jnp.maximum(a_v[i, sl] * svec[cb] + tvec[cb], 0.0)
                ae = jnp.exp(z - gvec[cb])
                mm_v[i, pl.ds(64 + cb * 16, 16)] = ae
                mm_v[i, sl] = ae * (hx_buf[i, sl] + d_v[i, sl])
            return carry2

        lax.fori_loop(0, KC, edge, 0, unroll=2)

    pltpu.sync_copy(src_h.at[pl.ds(base0, KC)], src_a)
    issue_gather(src_a, hx_a, sem_ga)

    def iot(g, carry):
        dst_a[pl.ds(g * 16, 16)] = lax.iota(jnp.int32, 16) + g * 16
        return carry

    lax.fori_loop(0, KC // 16, iot, 0)
    pltpu.async_copy(mm_v, acc.at[dst_a], sem_s, add=True)

    def pair(t, carry):
        j = 2 * t
        issue_linear(base0 + j * KC, dst_a)
        pltpu.sync_copy(src_h.at[pl.ds(base0 + (j + 1) * KC, KC)], src_b)
        issue_gather(src_b, hx_b, sem_gb)
        wait_linear(dst_a)
        wait_scatter()
        wait_gather(hx_a, sem_ga)
        edge_loop(hx_a)
        pltpu.async_copy(mm_v, acc.at[dst_a], sem_s, add=True)
        j1 = j + 1
        issue_linear(base0 + j1 * KC, dst_b)
        basen = jnp.minimum(base0 + (j1 + 1) * KC, base0 + (NCH - 1) * KC)
        pltpu.sync_copy(src_h.at[pl.ds(basen, KC)], src_a)
        issue_gather(src_a, hx_a, sem_ga)
        wait_linear(dst_b)
        wait_scatter()
        wait_gather(hx_b, sem_gb)
        edge_loop(hx_b)
        pltpu.async_copy(mm_v, acc.at[dst_b], sem_s, add=True)
        return carry

    lax.fori_loop(0, NCH // 2, pair, 0)
    wait_scatter()
    wait_gather(hx_a, sem_ga)
    plsc.subcore_barrier()
    pltpu.sync_copy(acc.at[pl.ds(row0, RPS)], acc_out.at[c, pl.ds(row0, RPS)])


def _s2(alL, alR, dlL, dlR, hxL, hxR, src, dst, coef):
    mesh = plsc.VectorSubcoreMesh(core_axis_name="c", subcore_axis_name="s")
    f32 = jnp.float32
    k = functools.partial(
        pl.kernel, mesh=mesh,
        out_type=[jax.ShapeDtypeStruct((2, N_ACC, D), f32)],
        scratch_types=[pltpu.VMEM((KC, 64), f32), pltpu.VMEM((KC, 64), f32),
                       pltpu.VMEM((KC, 64), f32), pltpu.VMEM((KC, 64), f32),
                       pltpu.VMEM((KC, D), f32),
                       pltpu.VMEM((KC,), jnp.int32), pltpu.VMEM((KC,), jnp.int32),
                       pltpu.VMEM((KC,), jnp.int32), pltpu.VMEM((KC,), jnp.int32),
                       pltpu.VMEM((3, 64), f32),
                       pltpu.VMEM_SHARED((N_ACC, D), f32),
                       pltpu.SemaphoreType.DMA, pltpu.SemaphoreType.DMA,
                       pltpu.SemaphoreType.DMA, pltpu.SemaphoreType.DMA],
        compiler_params=pltpu.CompilerParams(needs_layout_passes=False,
                                             use_tc_tiling_on_sc=False),
    )(_s2_body)
    return k(alL, alR, dlL, dlR, hxL, hxR, src, dst, coef)


def _t0_body(x_ref, winT, b_in, wlinT, wdstT, wsrcT, wattnT, b_attn,
             hx_o, p_o, q_o):
    h = jax.nn.relu(jnp.dot(x_ref[...], winT[...],
                            preferred_element_type=jnp.float32) + b_in[...])
    hx = jnp.dot(h, wlinT[...], preferred_element_type=jnp.float32)
    p = jnp.dot(jnp.dot(h, wdstT[...], preferred_element_type=jnp.float32),
                wattnT[...], preferred_element_type=jnp.float32) + b_attn[...]
    q = jnp.dot(jnp.dot(h, wsrcT[...], preferred_element_type=jnp.float32),
                wattnT[...], preferred_element_type=jnp.float32)
    hx_o[...] = hx
    p_o[...] = p
    q_o[...] = q


def _t0(x, winT, b_in, wlinT, wdstT, wsrcT, wattnT, b_attn):
    B = 2000
    grid = N // B
    full = lambda s: pl.BlockSpec(s, lambda i: (0, 0))
    return pl.pallas_call(
        _t0_body,
        grid=(grid,),
        in_specs=[pl.BlockSpec((B, D), lambda i: (i, 0)),
                  full((D, D)), full((1, D)), full((D, D)), full((D, D)),
                  full((D, D)), full((D, D)), full((1, D))],
        out_specs=[pl.BlockSpec((B, D), lambda i: (i, 0)),
                   pl.BlockSpec((B, D), lambda i: (i, 0)),
                   pl.BlockSpec((B, D), lambda i: (i, 0))],
        out_shape=[jax.ShapeDtypeStruct((N, D), jnp.float32),
                   jax.ShapeDtypeStruct((N, D), jnp.float32),
                   jax.ShapeDtypeStruct((N, D), jnp.float32)],
    )(x, winT, b_in, wlinT, wdstT, wsrcT, wattnT, b_attn)


def _t1_body(pqL, pqR, pw, a_ref, c0_ref, wattnT, alL_o, alR_o, dlL_o, dlR_o, st_o):
    i = pl.program_id(0)
    a = a_ref[...]
    delta = jax.nn.relu(pw[:, 0:1] * a[0:1, :] + pw[:, 1:2] * a[1:2, :]
                        + pw[:, 2:3] * a[2:3, :] + c0_ref[...])
    pq = jnp.concatenate([pqL[...], pqR[...]], axis=1)
    alpha = pq + jnp.dot(delta, wattnT[...], preferred_element_type=jnp.float32)
    alL_o[...] = alpha[:, :64]
    alR_o[...] = alpha[:, 64:]
    dlL_o[...] = delta[:, :64]
    dlR_o[...] = delta[:, 64:]
    w = pw[:, 3:4]
    s0 = jnp.sum(w * alpha, axis=0)[None, :]
    s1 = jnp.sum(w * alpha * alpha, axis=0)[None, :]
    mask = w > 0.0
    mx = jnp.max(jnp.where(mask, alpha, -3e38), axis=0)[None, :]
    mn = jnp.min(jnp.where(mask, alpha, 3e38), axis=0)[None, :]

    @pl.when(i == 0)
    def _init():
        st_o[...] = jnp.concatenate(
            [jnp.zeros((2, D), jnp.float32),
             jnp.full((1, D), -3e38, jnp.float32),
             jnp.full((1, D), 3e38, jnp.float32),
             jnp.zeros((4, D), jnp.float32)], axis=0)

    acc = st_o[...]
    st_o[...] = jnp.concatenate(
        [acc[0:1] + s0, acc[1:2] + s1,
         jnp.maximum(acc[2:3], mx), jnp.minimum(acc[3:4], mn), acc[4:8]], axis=0)


def _t1(pqL, pqR, pw, A3, c0, wattnT):
    B = NB_T1
    grid = E_PAD // B
    full = lambda s: pl.BlockSpec(s, lambda i: (0, 0))
    eb64 = pl.BlockSpec((B, 64), lambda i: (i, 0))
    return pl.pallas_call(
        _t1_body,
        grid=(grid,),
        in_specs=[eb64, eb64, pl.BlockSpec((B, 4), lambda i: (i, 0)),
                  full((3, D)), full((1, D)), full((D, D))],
        out_specs=[eb64, eb64, eb64, eb64, full((8, D))],
        out_shape=[jax.ShapeDtypeStruct((E_PAD, 64), jnp.float32),
                   jax.ShapeDtypeStruct((E_PAD, 64), jnp.float32),
                   jax.ShapeDtypeStruct((E_PAD, 64), jnp.float32),
                   jax.ShapeDtypeStruct((E_PAD, 64), jnp.float32),
                   jax.ShapeDtypeStruct((8, D), jnp.float32)],
    )(pqL, pqR, pw, A3, c0, wattnT)


def _t2_body(nF, aF, woutT, b_out, o_ref):
    outp = nF[...] / (aF[...] + 1e-16)
    o_ref[...] = jax.nn.relu(
        jnp.dot(outp, woutT[...], preferred_element_type=jnp.float32) + b_out[...])


def _t2(numerF, asumF, woutT, b_out):
    B = 2000
    grid = N // B
    full = lambda s: pl.BlockSpec(s, lambda i: (0, 0))
    return pl.pallas_call(
        _t2_body,
        grid=(grid,),
        in_specs=[pl.BlockSpec((B, D), lambda i: (i, 0)),
                  pl.BlockSpec((B, D), lambda i: (i, 0)),
                  full((D, D)), full((1, D))],
        out_specs=pl.BlockSpec((B, D), lambda i: (i, 0)),
        out_shape=jax.ShapeDtypeStruct((N, D), jnp.float32),
    )(numerF, asumF, woutT, b_out)


def kernel(x, pos, edge_index, W_in, b_in, W_out, b_out, W_pos, b_pos, g_pos,
           bt_pos, W_attn, b_attn, g_attn, bt_attn, W_lin, W_src, W_dst):
    E_orig = edge_index.shape[1]
    E_tot = E_orig + N
    pad = E_PAD - E_tot

    src_o, dst_o = edge_index[0], edge_index[1]
    keep_o = src_o != dst_o
    loop = jnp.arange(N, dtype=jnp.int32)
    src = jnp.concatenate([src_o, loop, jnp.zeros((pad,), jnp.int32)])
    dst = jnp.concatenate([jnp.where(keep_o, dst_o, N), loop,
                           jnp.full((pad,), N, jnp.int32)])
    w = jnp.concatenate([keep_o.astype(jnp.float32), jnp.ones((N,), jnp.float32),
                         jnp.zeros((pad,), jnp.float32)])
    dstc = jnp.minimum(dst, N - 1)

    hx, P, Q = _t0(x, W_in.T, b_in[None, :], W_lin.T, W_dst.T, W_src.T,
                   W_attn.T, b_attn[None, :])

    posx, posy, posz = pos[:, 0], pos[:, 1], pos[:, 2]
    pqL, pqR, pdx, pdy, pdz, mom = _s1(src, dstc, w, posx, posy, posz, P, Q)
    msum = jnp.sum(mom, axis=(0, 2))
    cnt = msum[9]
    S1v = msum[0:3]
    S2m = jnp.stack([jnp.stack([msum[3], msum[6], msum[7]]),
                     jnp.stack([msum[6], msum[4], msum[8]]),
                     jnp.stack([msum[7], msum[8], msum[5]])])

    pbar = S1v / cnt
    M = S2m / cnt - jnp.outer(pbar, pbar)
    var_pos = jnp.einsum("ca,ab,cb->c", W_pos, M, W_pos)
    s_pos = g_pos / jnp.sqrt(var_pos + 1e-5)
    A3 = W_pos.T * s_pos[None, :]
    c0 = bt_pos - (W_pos @ pbar) * s_pos

    pw = jnp.stack([pdx, pdy, pdz, w], axis=1)

    alL, alR, dlL, dlR, st = _t1(pqL, pqR, pw, A3, c0[None, :], W_attn.T)

    mu = st[0] / cnt
    var_a = st[1] / cnt - mu * mu
    s_att = g_attn / jnp.sqrt(var_a + 1e-5)
    t0 = bt_attn - mu * s_att
    gmax = jax.nn.relu(jnp.maximum(s_att * st[2], s_att * st[3]) + t0)

    coef = jnp.stack([jnp.stack([s_att[:64], t0[:64], gmax[:64]]),
                      jnp.stack([s_att[64:], t0[64:], gmax[64:]])])
    hxL = hx[:, :64] + 0.0
    hxR = hx[:, 64:] + 0.0
    (acc,) = _s2(alL, alR, dlL, dlR, hxL, hxR, src, dst, coef)
    numerF = jnp.concatenate([acc[0, :N, :64], acc[1, :N, :64]], axis=1)
    asumF = jnp.concatenate([acc[0, :N, 64:], acc[1, :N, 64:]], axis=1)

    return _t2(numerF, asumF, W_out.T, b_out[None, :])
